# baseline (speedup 1.0000x reference)
import jax
import jax.numpy as jnp
from jax import lax
from jax.experimental import pallas as pl
from jax.experimental.pallas import tpu as pltpu
from jax.experimental.pallas import tpu_sc as plsc

_B, _P, _C, _H, _W = 4, 14, 192, 16, 16
_NPATCH = _B * _P * _P
_CK = 16
_NCHUNK = _C // _CK
_JOBS = _NPATCH * _NCHUNK
_NC, _NS = 2, 16
_NW = _NC * _NS
_JPW = _JOBS // _NW


def _splat(v):
    return jnp.full((16,), v, jnp.int32)


def _body(x, tw, bw, lw, rw, tlw, trw, blw, brw, out, wts, xbuf, buf):
    wid = lax.axis_index("s") * _NC + lax.axis_index("c")
    pltpu.sync_copy(tw, wts.at[0])
    pltpu.sync_copy(bw, wts.at[1])
    pltpu.sync_copy(lw, wts.at[2])
    pltpu.sync_copy(rw, wts.at[3])
    pltpu.sync_copy(tlw, wts.at[4])
    pltpu.sync_copy(trw, wts.at[5])
    pltpu.sync_copy(blw, wts.at[6])
    pltpu.sync_copy(brw, wts.at[7])

    lanes = lax.iota(jnp.int32, 16)
    zeros = _splat(0)

    def job(j, carry):
        jg = wid * _JPW + j
        b = jg // _NCHUNK
        c0 = (jg % _NCHUNK) * _CK
        pr = (b % (_P * _P)) // _P
        pc = b % _P
        one = jnp.float32(1.0)
        zero = jnp.float32(0.0)
        mt = jnp.where(pr == 0, zero, one)
        mb = jnp.where(pr == _P - 1, zero, one)
        ml = jnp.where(pc == 0, zero, one)
        mr = jnp.where(pc == _P - 1, zero, one)

        pltpu.sync_copy(x.at[b, pl.ds(c0, _CK)], xbuf)
        for k in range(_CK):
            ck = _splat(k)
            for h in range(_H):
                r = xbuf[k, h]
                plsc.store_scatter(buf, [ck, _splat(h + 1), lanes + 1], r)

        twv = wts[0, pl.ds(c0, _CK)] * mt
        bwv = wts[1, pl.ds(c0, _CK)] * mb
        lwv = wts[2, pl.ds(c0, _CK)] * ml
        rwv = wts[3, pl.ds(c0, _CK)] * mr
        tlv = wts[4, pl.ds(c0, _CK)] * (mt * ml)
        trv = wts[5, pl.ds(c0, _CK)] * (mt * mr)
        blv = wts[6, pl.ds(c0, _CK)] * (mb * ml)
        brv = wts[7, pl.ds(c0, _CK)] * (mb * mr)

        for w in range(_W):
            cw = _splat(w + 1)
            g0 = plsc.load_gather(buf, [lanes, _splat(1), cw])
            g1 = plsc.load_gather(buf, [lanes, _splat(2), cw])
            plsc.store_scatter(buf, [lanes, zeros, cw], twv * (g0 + g1))
            g0 = plsc.load_gather(buf, [lanes, _splat(_H - 1), cw])
            g1 = plsc.load_gather(buf, [lanes, _splat(_H), cw])
            plsc.store_scatter(buf, [lanes, _splat(_H + 1), cw],
                               bwv * (g0 + g1))
        for h in range(_H):
            rh = _splat(h + 1)
            g0 = plsc.load_gather(buf, [lanes, rh, _splat(1)])
            g1 = plsc.load_gather(buf, [lanes, rh, _splat(2)])
            plsc.store_scatter(buf, [lanes, rh, zeros], lwv * (g0 + g1))
            g0 = plsc.load_gather(buf, [lanes, rh, _splat(_W - 1)])
            g1 = plsc.load_gather(buf, [lanes, rh, _splat(_W)])
            plsc.store_scatter(buf, [lanes, rh, _splat(_W + 1)],
                               rwv * (g0 + g1))

        g = plsc.load_gather(buf, [lanes, _splat(1), _splat(1)])
        plsc.store_scatter(buf, [lanes, zeros, zeros], tlv * g)
        g = plsc.load_gather(buf, [lanes, _splat(1), _splat(_W)])
        plsc.store_scatter(buf, [lanes, zeros, _splat(_W + 1)], trv * g)
        g = plsc.load_gather(buf, [lanes, _splat(_H), _splat(1)])
        plsc.store_scatter(buf, [lanes, _splat(_H + 1), zeros], blv * g)
        g = plsc.load_gather(buf, [lanes, _splat(_H), _splat(_W)])
        plsc.store_scatter(buf, [lanes, _splat(_H + 1), _splat(_W + 1)],
                           brv * g)

        pltpu.sync_copy(buf, out.at[b, pl.ds(c0, _CK)])
        return carry

    lax.fori_loop(0, _JPW, job, 0)


def kernel(x, topW, botW, leftW, rightW, topleftW, toprightW, botleftW,
           botrightW):
    mesh = plsc.VectorSubcoreMesh(core_axis_name="c", subcore_axis_name="s",
                                  num_cores=_NC, num_subcores=_NS)
    f = pl.kernel(
        _body,
        out_type=jax.ShapeDtypeStruct((_NPATCH, _C, _H + 2, _W + 2),
                                      jnp.float32),
        mesh=mesh,
        compiler_params=pltpu.CompilerParams(needs_layout_passes=False,
                                             use_tc_tiling_on_sc=False),
        scratch_types=[
            pltpu.VMEM((8, _C), jnp.float32),
            pltpu.VMEM((_CK, _H, _W), jnp.float32),
            pltpu.VMEM((_CK, _H + 2, _W + 2), jnp.float32),
        ],
    )
    return f(x, topW, botW, leftW, rightW, topleftW, toprightW, botleftW,
             botrightW)


# speedup vs baseline: 4.7780x; 4.7780x over previous
import jax
import jax.numpy as jnp
from jax import lax
from jax.experimental import pallas as pl
from jax.experimental.pallas import tpu as pltpu
from jax.experimental.pallas import tpu_sc as plsc

_B, _P, _C, _H, _W = 4, 14, 192, 16, 16
_NPATCH = _B * _P * _P
_CK = 16
_NCHUNK = _C // _CK
_JOBS = _NPATCH * _NCHUNK
_NC, _NS = 2, 16
_NW = _NC * _NS
_JPW = _JOBS // _NW


def _splat(v):
    return jnp.full((16,), v, jnp.int32)


def _body(x, tw, bw, lw, rw, tlw, trw, blw, brw, out, wts, xbuf, obuf):
    wid = lax.axis_index("s") * _NC + lax.axis_index("c")
    pltpu.sync_copy(tw, wts.at[0])
    pltpu.sync_copy(bw, wts.at[1])
    pltpu.sync_copy(lw, wts.at[2])
    pltpu.sync_copy(rw, wts.at[3])
    pltpu.sync_copy(tlw, wts.at[4])
    pltpu.sync_copy(trw, wts.at[5])
    pltpu.sync_copy(blw, wts.at[6])
    pltpu.sync_copy(brw, wts.at[7])

    lanes = lax.iota(jnp.int32, 16)

    def job(j, carry):
        jg = wid * _JPW + j
        b = jg // _NCHUNK
        c0 = (jg % _NCHUNK) * _CK
        pr = (b % (_P * _P)) // _P
        pc = b % _P
        one = jnp.float32(1.0)
        zero = jnp.float32(0.0)
        mt = jnp.where(pr == 0, zero, one)
        mb = jnp.where(pr == _P - 1, zero, one)
        ml = jnp.where(pc == 0, zero, one)
        mr = jnp.where(pc == _P - 1, zero, one)

        pltpu.sync_copy(x.at[b, pl.ds(c0, _CK)], xbuf)

        twv = wts[0, pl.ds(c0, _CK)] * mt
        bwv = wts[1, pl.ds(c0, _CK)] * mb
        lwv = wts[2, pl.ds(c0, _CK)] * ml
        rwv = wts[3, pl.ds(c0, _CK)] * mr
        tlv = wts[4, pl.ds(c0, _CK)] * (mt * ml)
        trv = wts[5, pl.ds(c0, _CK)] * (mt * mr)
        blv = wts[6, pl.ds(c0, _CK)] * (mb * ml)
        brv = wts[7, pl.ds(c0, _CK)] * (mb * mr)

        for w in range(_W):
            cw = _splat(w)
            g = [None] * _H
            for h in range(_H):
                g[h] = plsc.load_gather(xbuf, [lanes, _splat(h), cw])
                obuf[h + 1, w + 1] = g[h]
            obuf[0, w + 1] = twv * (g[0] + g[1])
            obuf[_H + 1, w + 1] = bwv * (g[_H - 2] + g[_H - 1])
        for h in range(_H):
            rh = _splat(h)
            g0 = plsc.load_gather(xbuf, [lanes, rh, _splat(0)])
            g1 = plsc.load_gather(xbuf, [lanes, rh, _splat(1)])
            obuf[h + 1, 0] = lwv * (g0 + g1)
            g0 = plsc.load_gather(xbuf, [lanes, rh, _splat(_W - 2)])
            g1 = plsc.load_gather(xbuf, [lanes, rh, _splat(_W - 1)])
            obuf[h + 1, _W + 1] = rwv * (g0 + g1)
        obuf[0, 0] = tlv * plsc.load_gather(xbuf, [lanes, _splat(0), _splat(0)])
        obuf[0, _W + 1] = trv * plsc.load_gather(
            xbuf, [lanes, _splat(0), _splat(_W - 1)])
        obuf[_H + 1, 0] = blv * plsc.load_gather(
            xbuf, [lanes, _splat(_H - 1), _splat(0)])
        obuf[_H + 1, _W + 1] = brv * plsc.load_gather(
            xbuf, [lanes, _splat(_H - 1), _splat(_W - 1)])

        pltpu.sync_copy(obuf, out.at[b, :, :, pl.ds(c0, _CK)])
        return carry

    lax.fori_loop(0, _JPW, job, 0)


def kernel(x, topW, botW, leftW, rightW, topleftW, toprightW, botleftW,
           botrightW):
    mesh = plsc.VectorSubcoreMesh(core_axis_name="c", subcore_axis_name="s",
                                  num_cores=_NC, num_subcores=_NS)
    f = pl.kernel(
        _body,
        out_type=jax.ShapeDtypeStruct((_NPATCH, _H + 2, _W + 2, _C),
                                      jnp.float32),
        mesh=mesh,
        compiler_params=pltpu.CompilerParams(needs_layout_passes=False,
                                             use_tc_tiling_on_sc=False),
        scratch_types=[
            pltpu.VMEM((8, _C), jnp.float32),
            pltpu.VMEM((_CK, _H, _W), jnp.float32),
            pltpu.VMEM((_H + 2, _W + 2, _CK), jnp.float32),
        ],
    )
    out = f(x, topW, botW, leftW, rightW, topleftW, toprightW, botleftW,
            botrightW)
    return jnp.transpose(out, (0, 3, 1, 2))


# speedup vs baseline: 5.0910x; 1.0655x over previous
import jax
import jax.numpy as jnp
from jax import lax
from jax.experimental import pallas as pl
from jax.experimental.pallas import tpu as pltpu
from jax.experimental.pallas import tpu_sc as plsc

_B, _P, _C, _H, _W = 4, 14, 192, 16, 16
_NPATCH = _B * _P * _P
_CK = 16
_NCHUNK = _C // _CK
_JOBS = _NPATCH * _NCHUNK
_NC, _NS = 2, 16
_NW = _NC * _NS
_JPW = _JOBS // _NW
_XT = _H * _W


def _body(xf, tw, bw, lw, rw, tlw, trw, blw, brw, out, wts,
          xb0, xb1, ob0, ob1, si0, si1, so0, so1):
    wid = lax.axis_index("s") * _NC + lax.axis_index("c")
    pltpu.sync_copy(tw, wts.at[0])
    pltpu.sync_copy(bw, wts.at[1])
    pltpu.sync_copy(lw, wts.at[2])
    pltpu.sync_copy(rw, wts.at[3])
    pltpu.sync_copy(tlw, wts.at[4])
    pltpu.sync_copy(trw, wts.at[5])
    pltpu.sync_copy(blw, wts.at[6])
    pltpu.sync_copy(brw, wts.at[7])

    lanes = lax.iota(jnp.int32, 16)
    ibase = lanes * _XT

    xbufs = (xb0, xb1)
    isems = (si0, si1)
    osems = (so0, so1)

    def src_slice(j):
        jg = wid * _JPW + j
        b = jg // _NCHUNK
        c0 = (jg % _NCHUNK) * _CK
        return xf.at[pl.ds((b * _C + c0) * _XT, _CK * _XT)]

    def start_in(j, p):
        pltpu.make_async_copy(src_slice(j), xbufs[p], isems[p]).start()

    def compute(j, p, obuf):
        jg = wid * _JPW + j
        b = jg // _NCHUNK
        c0 = (jg % _NCHUNK) * _CK
        xbuf = xbufs[p]
        pr = (b % (_P * _P)) // _P
        pc = b % _P
        one = jnp.float32(1.0)
        zero = jnp.float32(0.0)
        mt = jnp.where(pr == 0, zero, one)
        mb = jnp.where(pr == _P - 1, zero, one)
        ml = jnp.where(pc == 0, zero, one)
        mr = jnp.where(pc == _P - 1, zero, one)

        twv = wts[0, pl.ds(c0, _CK)] * mt
        bwv = wts[1, pl.ds(c0, _CK)] * mb
        lwv = wts[2, pl.ds(c0, _CK)] * ml
        rwv = wts[3, pl.ds(c0, _CK)] * mr
        tlv = wts[4, pl.ds(c0, _CK)] * (mt * ml)
        trv = wts[5, pl.ds(c0, _CK)] * (mt * mr)
        blv = wts[6, pl.ds(c0, _CK)] * (mb * ml)
        brv = wts[7, pl.ds(c0, _CK)] * (mb * mr)

        for w in range(_W):
            g0 = g1 = None
            for h in range(_H):
                g = plsc.load_gather(xbuf, [ibase + (16 * h + w)])
                obuf[h + 1, w + 1] = g
                if h == 0:
                    g0 = g
                elif h == 1:
                    g1 = g
                    obuf[0, w + 1] = twv * (g0 + g1)
                elif h == _H - 2:
                    g0 = g
                elif h == _H - 1:
                    obuf[_H + 1, w + 1] = bwv * (g0 + g)
        for h in range(_H):
            g0 = plsc.load_gather(xbuf, [ibase + 16 * h])
            g1 = plsc.load_gather(xbuf, [ibase + (16 * h + 1)])
            obuf[h + 1, 0] = lwv * (g0 + g1)
            g0 = plsc.load_gather(xbuf, [ibase + (16 * h + _W - 2)])
            g1 = plsc.load_gather(xbuf, [ibase + (16 * h + _W - 1)])
            obuf[h + 1, _W + 1] = rwv * (g0 + g1)
        obuf[0, 0] = tlv * plsc.load_gather(xbuf, [ibase])
        obuf[0, _W + 1] = trv * plsc.load_gather(xbuf, [ibase + (_W - 1)])
        obuf[_H + 1, 0] = blv * plsc.load_gather(xbuf, [ibase + (_XT - _W)])
        obuf[_H + 1, _W + 1] = brv * plsc.load_gather(
            xbuf, [ibase + (_XT - 1)])

    def out_copy(j, obuf, sem):
        jg = wid * _JPW + j
        b = jg // _NCHUNK
        c0 = (jg % _NCHUNK) * _CK
        return pltpu.make_async_copy(
            obuf, out.at[b, :, :, pl.ds(c0, _CK)], sem)

    start_in(0, 0)

    def iteration(j2, carry):
        for p in range(2):
            j = j2 + p
            pltpu.make_async_copy(src_slice(j), xbufs[p], isems[p]).wait()
            jn = jnp.minimum(j + 1, _JPW - 1)
            start_in(jn, 1 - p)

            @pl.when(j2 > 0)
            def _():
                out_copy(j - 2, (ob0, ob1)[p], osems[p]).wait()

            compute(j, p, (ob0, ob1)[p])
            out_copy(j, (ob0, ob1)[p], osems[p]).start()
        return carry

    lax.fori_loop(0, _JPW // 2, lambda i, c: iteration(i * 2, c), 0)

    out_copy(_JPW - 2, ob0, osems[0]).wait()
    out_copy(_JPW - 1, ob1, osems[1]).wait()
    pltpu.make_async_copy(src_slice(_JPW - 1), xbufs[0], isems[0]).wait()


def kernel(x, topW, botW, leftW, rightW, topleftW, toprightW, botleftW,
           botrightW):
    mesh = plsc.VectorSubcoreMesh(core_axis_name="c", subcore_axis_name="s",
                                  num_cores=_NC, num_subcores=_NS)
    f = pl.kernel(
        _body,
        out_type=jax.ShapeDtypeStruct((_NPATCH, _H + 2, _W + 2, _C),
                                      jnp.float32),
        mesh=mesh,
        compiler_params=pltpu.CompilerParams(needs_layout_passes=False,
                                             use_tc_tiling_on_sc=False),
        scratch_types=[
            pltpu.VMEM((8, _C), jnp.float32),
            pltpu.VMEM((_CK * _XT,), jnp.float32),
            pltpu.VMEM((_CK * _XT,), jnp.float32),
            pltpu.VMEM((_H + 2, _W + 2, _CK), jnp.float32),
            pltpu.VMEM((_H + 2, _W + 2, _CK), jnp.float32),
            pltpu.SemaphoreType.DMA,
            pltpu.SemaphoreType.DMA,
            pltpu.SemaphoreType.DMA,
            pltpu.SemaphoreType.DMA,
        ],
    )
    out = f(x.reshape(-1), topW, botW, leftW, rightW, topleftW, toprightW,
            botleftW, botrightW)
    return jnp.transpose(out, (0, 3, 1, 2))


# speedup vs baseline: 6.1608x; 1.2101x over previous
import jax
import jax.numpy as jnp
from jax import lax
from jax.experimental import pallas as pl
from jax.experimental.pallas import tpu as pltpu
from jax.experimental.pallas import tpu_sc as plsc

_B, _P, _C, _H, _W = 4, 14, 192, 16, 16
_NPATCH = _B * _P * _P
_CK = 16
_NCHUNK = _C // _CK
_JOBS = _NPATCH * _NCHUNK
_NC, _NS = 2, 16
_NW = _NC * _NS
_JPW = _JOBS // _NW
_XT = _H * _W


def _body(xf, tw, bw, lw, rw, tlw, trw, blw, brw, out, wts,
          xb0, xb1, ob0, ob1, si0, si1, so0, so1):
    wid = lax.axis_index("s") * _NC + lax.axis_index("c")
    pltpu.sync_copy(tw, wts.at[0])
    pltpu.sync_copy(bw, wts.at[1])
    pltpu.sync_copy(lw, wts.at[2])
    pltpu.sync_copy(rw, wts.at[3])
    pltpu.sync_copy(tlw, wts.at[4])
    pltpu.sync_copy(trw, wts.at[5])
    pltpu.sync_copy(blw, wts.at[6])
    pltpu.sync_copy(brw, wts.at[7])

    lanes = lax.iota(jnp.int32, 16)
    ibase = lanes * _XT

    xbufs = (xb0, xb1)
    isems = (si0, si1)
    osems = (so0, so1)

    def src_slice(j):
        jg = wid * _JPW + j
        b = jg // _NCHUNK
        c0 = (jg % _NCHUNK) * _CK
        return xf.at[pl.ds((b * _C + c0) * _XT, _CK * _XT)]

    def start_in(j, p):
        pltpu.make_async_copy(src_slice(j), xbufs[p], isems[p]).start()

    def compute(j, p, obuf):
        jg = wid * _JPW + j
        b = jg // _NCHUNK
        c0 = (jg % _NCHUNK) * _CK
        xbuf = xbufs[p]
        pr = (b % (_P * _P)) // _P
        pc = b % _P
        one = jnp.float32(1.0)
        zero = jnp.float32(0.0)
        mt = jnp.where(pr == 0, zero, one)
        mb = jnp.where(pr == _P - 1, zero, one)
        ml = jnp.where(pc == 0, zero, one)
        mr = jnp.where(pc == _P - 1, zero, one)

        twv = wts[0, pl.ds(c0, _CK)] * mt
        bwv = wts[1, pl.ds(c0, _CK)] * mb
        lwv = wts[2, pl.ds(c0, _CK)] * ml
        rwv = wts[3, pl.ds(c0, _CK)] * mr
        tlv = wts[4, pl.ds(c0, _CK)] * (mt * ml)
        trv = wts[5, pl.ds(c0, _CK)] * (mt * mr)
        blv = wts[6, pl.ds(c0, _CK)] * (mb * ml)
        brv = wts[7, pl.ds(c0, _CK)] * (mb * mr)

        for w in range(_W):
            g = [plsc.load_gather(xbuf, [ibase + (16 * h + w)])
                 for h in range(_H)]
            for h in range(_H):
                obuf[h + 1, w + 1] = g[h]
            obuf[0, w + 1] = twv * (g[0] + g[1])
            obuf[_H + 1, w + 1] = bwv * (g[_H - 2] + g[_H - 1])
            if w == 0:
                gl = g
            elif w == 1:
                for h in range(_H):
                    obuf[h + 1, 0] = lwv * (gl[h] + g[h])
                obuf[0, 0] = tlv * gl[0]
                obuf[_H + 1, 0] = blv * gl[_H - 1]
                gl = None
            elif w == _W - 2:
                gl = g
            elif w == _W - 1:
                for h in range(_H):
                    obuf[h + 1, _W + 1] = rwv * (gl[h] + g[h])
                obuf[0, _W + 1] = trv * g[0]
                obuf[_H + 1, _W + 1] = brv * g[_H - 1]

    def out_copy(j, obuf, sem):
        jg = wid * _JPW + j
        b = jg // _NCHUNK
        c0 = (jg % _NCHUNK) * _CK
        return pltpu.make_async_copy(
            obuf, out.at[b, :, :, pl.ds(c0, _CK)], sem)

    start_in(0, 0)

    def iteration(j2, carry):
        for p in range(2):
            j = j2 + p
            pltpu.make_async_copy(src_slice(j), xbufs[p], isems[p]).wait()
            jn = jnp.minimum(j + 1, _JPW - 1)
            start_in(jn, 1 - p)

            @pl.when(j2 > 0)
            def _():
                out_copy(j - 2, (ob0, ob1)[p], osems[p]).wait()

            compute(j, p, (ob0, ob1)[p])
            out_copy(j, (ob0, ob1)[p], osems[p]).start()
        return carry

    lax.fori_loop(0, _JPW // 2, lambda i, c: iteration(i * 2, c), 0)

    out_copy(_JPW - 2, ob0, osems[0]).wait()
    out_copy(_JPW - 1, ob1, osems[1]).wait()
    pltpu.make_async_copy(src_slice(_JPW - 1), xbufs[0], isems[0]).wait()


def kernel(x, topW, botW, leftW, rightW, topleftW, toprightW, botleftW,
           botrightW):
    mesh = plsc.VectorSubcoreMesh(core_axis_name="c", subcore_axis_name="s",
                                  num_cores=_NC, num_subcores=_NS)
    f = pl.kernel(
        _body,
        out_type=jax.ShapeDtypeStruct((_NPATCH, _H + 2, _W + 2, _C),
                                      jnp.float32),
        mesh=mesh,
        compiler_params=pltpu.CompilerParams(needs_layout_passes=False,
                                             use_tc_tiling_on_sc=False),
        scratch_types=[
            pltpu.VMEM((8, _C), jnp.float32),
            pltpu.VMEM((_CK * _XT,), jnp.float32),
            pltpu.VMEM((_CK * _XT,), jnp.float32),
            pltpu.VMEM((_H + 2, _W + 2, _CK), jnp.float32),
            pltpu.VMEM((_H + 2, _W + 2, _CK), jnp.float32),
            pltpu.SemaphoreType.DMA,
            pltpu.SemaphoreType.DMA,
            pltpu.SemaphoreType.DMA,
            pltpu.SemaphoreType.DMA,
        ],
    )
    out = f(x.reshape(-1), topW, botW, leftW, rightW, topleftW, toprightW,
            botleftW, botrightW)
    return jnp.transpose(out, (0, 3, 1, 2))


# speedup vs baseline: 22.3394x; 3.6261x over previous
import jax
import jax.numpy as jnp
from jax import lax
from jax.experimental import pallas as pl
from jax.experimental.pallas import tpu as pltpu
from jax.experimental.pallas import tpu_sc as plsc

_B, _P, _C, _H, _W = 4, 14, 192, 16, 16
_NPATCH = _B * _P * _P
_BK = 16
_NBCHUNK = _NPATCH // _BK
_JOBS = _C * _NBCHUNK
_NC, _NS = 2, 16
_NW = _NC * _NS
_JPW = _JOBS // _NW


def _splat(v):
    return jnp.full((16,), v, jnp.int32)


def _body(xp, tw, bw, lw, rw, tlw, trw, blw, brw, out, wts,
          xb0, xb1, ob0, ob1, si0, si1, so0, so1):
    wid = lax.axis_index("s") * _NC + lax.axis_index("c")
    pltpu.sync_copy(tw, wts.at[0])
    pltpu.sync_copy(bw, wts.at[1])
    pltpu.sync_copy(lw, wts.at[2])
    pltpu.sync_copy(rw, wts.at[3])
    pltpu.sync_copy(tlw, wts.at[4])
    pltpu.sync_copy(trw, wts.at[5])
    pltpu.sync_copy(blw, wts.at[6])
    pltpu.sync_copy(brw, wts.at[7])

    lanes = lax.iota(jnp.int32, 16)

    xbufs = (xb0, xb1)
    isems = (si0, si1)
    osems = (so0, so1)

    def decode(j):
        jg = wid * _JPW + j
        return jg // _NBCHUNK, (jg % _NBCHUNK) * _BK

    def src_slice(j):
        c, b0 = decode(j)
        return xp.at[c, :, :, pl.ds(b0, _BK)]

    def start_in(j, p):
        pltpu.make_async_copy(src_slice(j), xbufs[p], isems[p]).start()

    def compute(j, p, obuf):
        c, b0 = decode(j)
        xbuf = xbufs[p]
        bv = b0 + lanes
        pq = bv % (_P * _P)
        pr = pq // _P
        pc = pq % _P
        onev = jnp.full((16,), 1.0, jnp.float32)
        zerov = jnp.zeros((16,), jnp.float32)
        mt = jnp.where(pr == 0, zerov, onev)
        mb = jnp.where(pr == _P - 1, zerov, onev)
        ml = jnp.where(pc == 0, zerov, onev)
        mr = jnp.where(pc == _P - 1, zerov, onev)

        ci = _splat(c)
        twv = plsc.load_gather(wts, [_splat(0), ci]) * mt
        bwv = plsc.load_gather(wts, [_splat(1), ci]) * mb
        lwv = plsc.load_gather(wts, [_splat(2), ci]) * ml
        rwv = plsc.load_gather(wts, [_splat(3), ci]) * mr
        tlv = plsc.load_gather(wts, [_splat(4), ci]) * (mt * ml)
        trv = plsc.load_gather(wts, [_splat(5), ci]) * (mt * mr)
        blv = plsc.load_gather(wts, [_splat(6), ci]) * (mb * ml)
        brv = plsc.load_gather(wts, [_splat(7), ci]) * (mb * mr)

        gl = None
        for w in range(_W):
            g = [xbuf[h, w] for h in range(_H)]
            for h in range(_H):
                obuf[h + 1, w + 1] = g[h]
            obuf[0, w + 1] = twv * (g[0] + g[1])
            obuf[_H + 1, w + 1] = bwv * (g[_H - 2] + g[_H - 1])
            if w == 0:
                gl = g
            elif w == 1:
                for h in range(_H):
                    obuf[h + 1, 0] = lwv * (gl[h] + g[h])
                obuf[0, 0] = tlv * gl[0]
                obuf[_H + 1, 0] = blv * gl[_H - 1]
            elif w == _W - 2:
                gl = g
            elif w == _W - 1:
                for h in range(_H):
                    obuf[h + 1, _W + 1] = rwv * (gl[h] + g[h])
                obuf[0, _W + 1] = trv * g[0]
                obuf[_H + 1, _W + 1] = brv * g[_H - 1]

    def out_copy(j, obuf, sem):
        c, b0 = decode(j)
        return pltpu.make_async_copy(
            obuf, out.at[:, :, c, pl.ds(b0, _BK)], sem)

    start_in(0, 0)

    def iteration(j2, carry):
        for p in range(2):
            j = j2 + p
            pltpu.make_async_copy(src_slice(j), xbufs[p], isems[p]).wait()
            jn = jnp.minimum(j + 1, _JPW - 1)
            start_in(jn, 1 - p)

            @pl.when(j2 > 0)
            def _():
                out_copy(j - 2, (ob0, ob1)[p], osems[p]).wait()

            compute(j, p, (ob0, ob1)[p])
            out_copy(j, (ob0, ob1)[p], osems[p]).start()
        return carry

    lax.fori_loop(0, _JPW // 2, lambda i, c: iteration(i * 2, c), 0)

    out_copy(_JPW - 2, ob0, osems[0]).wait()
    out_copy(_JPW - 1, ob1, osems[1]).wait()
    pltpu.make_async_copy(src_slice(_JPW - 1), xbufs[0], isems[0]).wait()


def kernel(x, topW, botW, leftW, rightW, topleftW, toprightW, botleftW,
           botrightW):
    mesh = plsc.VectorSubcoreMesh(core_axis_name="c", subcore_axis_name="s",
                                  num_cores=_NC, num_subcores=_NS)
    f = pl.kernel(
        _body,
        out_type=jax.ShapeDtypeStruct((_H + 2, _W + 2, _C, _NPATCH),
                                      jnp.float32),
        mesh=mesh,
        compiler_params=pltpu.CompilerParams(needs_layout_passes=False,
                                             use_tc_tiling_on_sc=False),
        scratch_types=[
            pltpu.VMEM((8, _C), jnp.float32),
            pltpu.VMEM((_H, _W, _BK), jnp.float32),
            pltpu.VMEM((_H, _W, _BK), jnp.float32),
            pltpu.VMEM((_H + 2, _W + 2, _BK), jnp.float32),
            pltpu.VMEM((_H + 2, _W + 2, _BK), jnp.float32),
            pltpu.SemaphoreType.DMA,
            pltpu.SemaphoreType.DMA,
            pltpu.SemaphoreType.DMA,
            pltpu.SemaphoreType.DMA,
        ],
    )
    xp = jnp.transpose(x, (1, 2, 3, 0))
    op = f(xp, topW, botW, leftW, rightW, topleftW, toprightW, botleftW,
           botrightW)
    return jnp.transpose(op, (3, 2, 0, 1))


# speedup vs baseline: 31.0649x; 1.3906x over previous
import jax
import jax.numpy as jnp
from jax import lax
from jax.experimental import pallas as pl
from jax.experimental.pallas import tpu as pltpu
from jax.experimental.pallas import tpu_sc as plsc

_B, _P, _C, _H, _W = 4, 14, 192, 16, 16
_NPATCH = _B * _P * _P
_BK = 112
_NSUB = _BK // 16
_NBCHUNK = _NPATCH // _BK
_JOBS = _C * _NBCHUNK
_NC, _NS = 2, 16
_NW = _NC * _NS
_JPW = _JOBS // _NW


def _splat(v):
    return jnp.full((16,), v, jnp.int32)


def _body(xp, tw, bw, lw, rw, tlw, trw, blw, brw, out, wts,
          xb0, xb1, obuf, si0, si1, so):
    wid = lax.axis_index("s") * _NC + lax.axis_index("c")
    pltpu.sync_copy(tw, wts.at[0])
    pltpu.sync_copy(bw, wts.at[1])
    pltpu.sync_copy(lw, wts.at[2])
    pltpu.sync_copy(rw, wts.at[3])
    pltpu.sync_copy(tlw, wts.at[4])
    pltpu.sync_copy(trw, wts.at[5])
    pltpu.sync_copy(blw, wts.at[6])
    pltpu.sync_copy(brw, wts.at[7])

    lanes = lax.iota(jnp.int32, 16)

    xbufs = (xb0, xb1)
    isems = (si0, si1)

    def decode(j):
        jg = wid * _JPW + j
        return jg // _NBCHUNK, (jg % _NBCHUNK) * _BK

    def src_slice(j):
        c, b0 = decode(j)
        return xp.at[c, :, :, pl.ds(b0, _BK)]

    def start_in(j, p):
        pltpu.make_async_copy(src_slice(j), xbufs[p], isems[p]).start()

    def compute(j, p):
        c, b0 = decode(j)
        xbuf = xbufs[p]
        ci = _splat(c)
        wtc = plsc.load_gather(wts, [_splat(0), ci])
        wbc = plsc.load_gather(wts, [_splat(1), ci])
        wlc = plsc.load_gather(wts, [_splat(2), ci])
        wrc = plsc.load_gather(wts, [_splat(3), ci])
        wtl = plsc.load_gather(wts, [_splat(4), ci])
        wtr = plsc.load_gather(wts, [_splat(5), ci])
        wbl = plsc.load_gather(wts, [_splat(6), ci])
        wbr = plsc.load_gather(wts, [_splat(7), ci])
        onev = jnp.full((16,), 1.0, jnp.float32)
        zerov = jnp.zeros((16,), jnp.float32)

        def sub_body(sub, carry):
            s0 = sub * 16
            bv = (b0 + s0) + lanes
            pq = bv % (_P * _P)
            pr = pq // _P
            pc = pq % _P
            mt = jnp.where(pr == 0, zerov, onev)
            mb = jnp.where(pr == _P - 1, zerov, onev)
            ml = jnp.where(pc == 0, zerov, onev)
            mr = jnp.where(pc == _P - 1, zerov, onev)
            twv = wtc * mt
            bwv = wbc * mb
            lwv = wlc * ml
            rwv = wrc * mr
            tlv = wtl * (mt * ml)
            trv = wtr * (mt * mr)
            blv = wbl * (mb * ml)
            brv = wbr * (mb * mr)

            gl = None
            for w in range(_W):
                g = [xbuf[h, w, pl.ds(s0, 16)] for h in range(_H)]
                for h in range(_H):
                    obuf[h + 1, w + 1, pl.ds(s0, 16)] = g[h]
                obuf[0, w + 1, pl.ds(s0, 16)] = twv * (g[0] + g[1])
                obuf[_H + 1, w + 1, pl.ds(s0, 16)] = bwv * (
                    g[_H - 2] + g[_H - 1])
                if w == 0:
                    gl = g
                elif w == 1:
                    for h in range(_H):
                        obuf[h + 1, 0, pl.ds(s0, 16)] = lwv * (gl[h] + g[h])
                    obuf[0, 0, pl.ds(s0, 16)] = tlv * gl[0]
                    obuf[_H + 1, 0, pl.ds(s0, 16)] = blv * gl[_H - 1]
                elif w == _W - 2:
                    gl = g
                elif w == _W - 1:
                    for h in range(_H):
                        obuf[h + 1, _W + 1, pl.ds(s0, 16)] = rwv * (
                            gl[h] + g[h])
                    obuf[0, _W + 1, pl.ds(s0, 16)] = trv * g[0]
                    obuf[_H + 1, _W + 1, pl.ds(s0, 16)] = brv * g[_H - 1]
            return carry

        lax.fori_loop(0, _NSUB, sub_body, 0)

    def out_copy(j):
        c, b0 = decode(j)
        return pltpu.make_async_copy(
            obuf, out.at[:, :, c, pl.ds(b0, _BK)], so)

    start_in(0, 0)

    def iteration(j2, carry):
        for p in range(2):
            j = j2 + p
            pltpu.make_async_copy(src_slice(j), xbufs[p], isems[p]).wait()
            jn = jnp.minimum(j + 1, _JPW - 1)
            start_in(jn, 1 - p)

            if p == 0:
                @pl.when(j2 > 0)
                def _():
                    out_copy(j - 1).wait()
            else:
                out_copy(j - 1).wait()

            compute(j, p)
            out_copy(j).start()
        return carry

    lax.fori_loop(0, _JPW // 2, lambda i, c: iteration(i * 2, c), 0)

    out_copy(_JPW - 1).wait()
    pltpu.make_async_copy(src_slice(_JPW - 1), xbufs[0], isems[0]).wait()


def kernel(x, topW, botW, leftW, rightW, topleftW, toprightW, botleftW,
           botrightW):
    mesh = plsc.VectorSubcoreMesh(core_axis_name="c", subcore_axis_name="s",
                                  num_cores=_NC, num_subcores=_NS)
    f = pl.kernel(
        _body,
        out_type=jax.ShapeDtypeStruct((_H + 2, _W + 2, _C, _NPATCH),
                                      jnp.float32),
        mesh=mesh,
        compiler_params=pltpu.CompilerParams(needs_layout_passes=False,
                                             use_tc_tiling_on_sc=False),
        scratch_types=[
            pltpu.VMEM((8, _C), jnp.float32),
            pltpu.VMEM((_H, _W, _BK), jnp.float32),
            pltpu.VMEM((_H, _W, _BK), jnp.float32),
            pltpu.VMEM((_H + 2, _W + 2, _BK), jnp.float32),
            pltpu.SemaphoreType.DMA,
            pltpu.SemaphoreType.DMA,
            pltpu.SemaphoreType.DMA,
        ],
    )
    xp = jnp.transpose(x, (1, 2, 3, 0))
    op = f(xp, topW, botW, leftW, rightW, topleftW, toprightW, botleftW,
           botrightW)
    return jnp.transpose(op, (3, 2, 0, 1))


# speedup vs baseline: 31.2855x; 1.0071x over previous
import jax
import jax.numpy as jnp
from jax import lax
from jax.experimental import pallas as pl
from jax.experimental.pallas import tpu as pltpu
from jax.experimental.pallas import tpu_sc as plsc

_B, _P, _C, _H, _W = 4, 14, 192, 16, 16
_NPATCH = _B * _P * _P
_BK = 112
_NSUB = _BK // 16
_NBCHUNK = _NPATCH // _BK
_JOBS = _C * _NBCHUNK
_NC, _NS = 2, 16
_NW = _NC * _NS
_JPW = _JOBS // _NW
_NBUF = 3


def _splat(v):
    return jnp.full((16,), v, jnp.int32)


def _body(xp, tw, bw, lw, rw, tlw, trw, blw, brw, out, wts,
          ob0, ob1, ob2, si0, si1, si2, so0, so1, so2):
    wid = lax.axis_index("s") * _NC + lax.axis_index("c")
    pltpu.sync_copy(tw, wts.at[0])
    pltpu.sync_copy(bw, wts.at[1])
    pltpu.sync_copy(lw, wts.at[2])
    pltpu.sync_copy(rw, wts.at[3])
    pltpu.sync_copy(tlw, wts.at[4])
    pltpu.sync_copy(trw, wts.at[5])
    pltpu.sync_copy(blw, wts.at[6])
    pltpu.sync_copy(brw, wts.at[7])

    lanes = lax.iota(jnp.int32, 16)

    obufs = (ob0, ob1, ob2)
    isems = (si0, si1, si2)
    osems = (so0, so1, so2)

    def decode(j):
        jg = wid * _JPW + j
        return jg // _NBCHUNK, (jg % _NBCHUNK) * _BK

    def in_copy(j, p):
        c, b0 = decode(j)
        return pltpu.make_async_copy(
            xp.at[c, :, :, pl.ds(b0, _BK)],
            obufs[p].at[pl.ds(1, _H), pl.ds(1, _W), :], isems[p])

    def compute(j, p):
        c, b0 = decode(j)
        obuf = obufs[p]
        ci = _splat(c)
        wtc = plsc.load_gather(wts, [_splat(0), ci])
        wbc = plsc.load_gather(wts, [_splat(1), ci])
        wlc = plsc.load_gather(wts, [_splat(2), ci])
        wrc = plsc.load_gather(wts, [_splat(3), ci])
        wtl = plsc.load_gather(wts, [_splat(4), ci])
        wtr = plsc.load_gather(wts, [_splat(5), ci])
        wbl = plsc.load_gather(wts, [_splat(6), ci])
        wbr = plsc.load_gather(wts, [_splat(7), ci])
        onev = jnp.full((16,), 1.0, jnp.float32)
        zerov = jnp.zeros((16,), jnp.float32)

        def sub_body(sub, carry):
            s0 = sub * 16
            ds = pl.ds(s0, 16)
            bv = (b0 + s0) + lanes
            pq = bv % (_P * _P)
            pr = pq // _P
            pc = pq % _P
            mt = jnp.where(pr == 0, zerov, onev)
            mb = jnp.where(pr == _P - 1, zerov, onev)
            ml = jnp.where(pc == 0, zerov, onev)
            mr = jnp.where(pc == _P - 1, zerov, onev)
            twv = wtc * mt
            bwv = wbc * mb
            lwv = wlc * ml
            rwv = wrc * mr
            tlv = wtl * (mt * ml)
            trv = wtr * (mt * mr)
            blv = wbl * (mb * ml)
            brv = wbr * (mb * mr)

            r0 = [obuf[1, w + 1, ds] for w in range(_W)]
            r1 = [obuf[2, w + 1, ds] for w in range(_W)]
            for w in range(_W):
                obuf[0, w + 1, ds] = twv * (r0[w] + r1[w])
            r14 = [obuf[_H - 1, w + 1, ds] for w in range(_W)]
            r15 = [obuf[_H, w + 1, ds] for w in range(_W)]
            for w in range(_W):
                obuf[_H + 1, w + 1, ds] = bwv * (r14[w] + r15[w])
            c0 = [obuf[h + 1, 1, ds] for h in range(_H)]
            c1 = [obuf[h + 1, 2, ds] for h in range(_H)]
            for h in range(_H):
                obuf[h + 1, 0, ds] = lwv * (c0[h] + c1[h])
            c14 = [obuf[h + 1, _W - 1, ds] for h in range(_H)]
            c15 = [obuf[h + 1, _W, ds] for h in range(_H)]
            for h in range(_H):
                obuf[h + 1, _W + 1, ds] = rwv * (c14[h] + c15[h])
            obuf[0, 0, ds] = tlv * c0[0]
            obuf[0, _W + 1, ds] = trv * c15[0]
            obuf[_H + 1, 0, ds] = blv * c0[_H - 1]
            obuf[_H + 1, _W + 1, ds] = brv * c15[_H - 1]
            return carry

        lax.fori_loop(0, _NSUB, sub_body, 0)

    def out_copy(j, p):
        c, b0 = decode(j)
        return pltpu.make_async_copy(
            obufs[p], out.at[:, :, c, pl.ds(b0, _BK)], osems[p])

    in_copy(0, 0).start()

    def iteration(j3, carry):
        for p in range(_NBUF):
            j = j3 + p
            @pl.when(j >= 2)
            def _():
                out_copy(j - 2, (p + 1) % _NBUF).wait()

            @pl.when(j + 1 <= _JPW - 1)
            def _():
                in_copy(j + 1, (p + 1) % _NBUF).start()

            in_copy(j, p).wait()
            compute(j, p)
            out_copy(j, p).start()
        return carry

    lax.fori_loop(0, _JPW // _NBUF, lambda i, c: iteration(i * _NBUF, c), 0)

    out_copy(_JPW - 2, (_JPW - 2) % _NBUF).wait()
    out_copy(_JPW - 1, (_JPW - 1) % _NBUF).wait()


def kernel(x, topW, botW, leftW, rightW, topleftW, toprightW, botleftW,
           botrightW):
    mesh = plsc.VectorSubcoreMesh(core_axis_name="c", subcore_axis_name="s",
                                  num_cores=_NC, num_subcores=_NS)
    f = pl.kernel(
        _body,
        out_type=jax.ShapeDtypeStruct((_H + 2, _W + 2, _C, _NPATCH),
                                      jnp.float32),
        mesh=mesh,
        compiler_params=pltpu.CompilerParams(needs_layout_passes=False,
                                             use_tc_tiling_on_sc=False),
        scratch_types=[
            pltpu.VMEM((8, _C), jnp.float32),
            pltpu.VMEM((_H + 2, _W + 2, _BK), jnp.float32),
            pltpu.VMEM((_H + 2, _W + 2, _BK), jnp.float32),
            pltpu.VMEM((_H + 2, _W + 2, _BK), jnp.float32),
            pltpu.SemaphoreType.DMA,
            pltpu.SemaphoreType.DMA,
            pltpu.SemaphoreType.DMA,
            pltpu.SemaphoreType.DMA,
            pltpu.SemaphoreType.DMA,
            pltpu.SemaphoreType.DMA,
        ],
    )
    xp = jnp.transpose(x, (1, 2, 3, 0))
    op = f(xp, topW, botW, leftW, rightW, topleftW, toprightW, botleftW,
           botrightW)
    return jnp.transpose(op, (3, 2, 0, 1))
